# baseline (speedup 1.0000x reference)
import jax
import jax.numpy as jnp
from jax import lax
from jax.experimental import pallas as pl

B = 16384
C = 1000
ROWS = 256


def _body(cos_ref, phi_ref, lab_ref, out_ref):
    cos = cos_ref[...]
    phi = phi_ref[...]
    lab = lab_ref[0, 0, :]
    col = lax.broadcasted_iota(jnp.int32, (ROWS, C), 1)
    mask = col == lab[:, None]
    z = jnp.where(mask, phi, cos)
    phil = jnp.sum(jnp.where(mask, phi, 0.0), axis=1)
    m = jnp.max(z, axis=1)
    s = jnp.sum(jnp.exp(z - m[:, None]), axis=1)
    nll = m + jnp.log(s) - phil

    @pl.when(pl.program_id(0) == 0)
    def _():
        out_ref[...] = jnp.zeros((1, 1), jnp.float32)

    out_ref[...] += jnp.sum(nll).reshape(1, 1)


def kernel(cos_theta, phi_theta, labels):
    nb = B // ROWS
    lab3 = labels.reshape(nb, 1, ROWS)
    total = pl.pallas_call(
        _body,
        grid=(nb,),
        in_specs=[
            pl.BlockSpec((ROWS, C), lambda i: (i, 0)),
            pl.BlockSpec((ROWS, C), lambda i: (i, 0)),
            pl.BlockSpec((1, 1, ROWS), lambda i: (i, 0, 0)),
        ],
        out_specs=pl.BlockSpec((1, 1), lambda i: (0, 0)),
        out_shape=jax.ShapeDtypeStruct((1, 1), jnp.float32),
    )(cos_theta, phi_theta, lab3)
    return total[0, 0] / B
